# baseline (speedup 1.0000x reference)
import functools

import jax
import jax.numpy as jnp
from jax import lax
from jax.experimental import pallas as pl
from jax.experimental.pallas import tpu as pltpu
from jax.experimental.pallas import tpu_sc as plsc

NUM_SEG = 1024
NC = 2
NS = 16
NW = NC * NS
LANES = 16


def _lane_gather(x, idx):
    dnums = lax.GatherDimensionNumbers(
        offset_dims=(), collapsed_slice_dims=(0,), start_index_map=(0,))
    return lax.gather(x, idx[:, None], dnums, (1,),
                      mode=lax.GatherScatterMode.PROMISE_IN_BOUNDS)


def _sc_segment_partials(n_rows: int):
    rows_w = n_rows // NW
    assert rows_w * NW == n_rows
    nb = 10
    br = rows_w // nb
    assert br * nb == rows_w and br % 8 == 0
    cr = 400
    nch = br // cr
    assert nch * cr == br and cr % LANES == 0
    kv = cr // LANES

    def body(dx, dy, ids, out_s, out_c,
             bufx0, bufy0, bufx1, bufy1, bufi0, bufi1,
             loc_s, loc_c, accx, accy, cur, cnt,
             sd0, si0, sd1, si1):
        w = lax.axis_index("s") * NC + lax.axis_index("c")
        row0 = w * rows_w
        zero16 = jnp.zeros((LANES,), jnp.float32)
        iota = lax.iota(jnp.int32, LANES)
        lane0 = iota == 0
        lane01 = iota < 2

        def z_s(i, c):
            loc_s[pl.ds(i * LANES, LANES)] = zero16
            return c

        lax.fori_loop(0, (2 * NUM_SEG) // LANES, z_s, 0)

        def z_c(i, c):
            loc_c[pl.ds(i * LANES, LANES)] = zero16
            return c

        lax.fori_loop(0, NUM_SEG // LANES, z_c, 0)
        accx[...] = zero16
        accy[...] = zero16
        cnt[0] = 0
        cur[0] = 0

        def start(blk, bx, by, bi, semd, semi):
            r = pl.ds(row0 + blk * br, br)
            pltpu.async_copy(dx.at[r], bx, semd)
            pltpu.async_copy(dy.at[r], by, semd)
            pltpu.async_copy(ids.at[r], bi, semi)

        def wait(blk, bx, by, bi, semd, semi):
            r = pl.ds(row0 + blk * br, br)
            pltpu.make_async_copy(dx.at[r], bx, semd).wait()
            pltpu.make_async_copy(dy.at[r], by, semd).wait()
            pltpu.make_async_copy(ids.at[r], bi, semi).wait()

        def flush(c):
            bx = accx[...]
            by = accy[...]
            for d in (1, 2, 4, 8):
                bx = bx + _lane_gather(bx, iota ^ d)
                by = by + _lane_gather(by, iota ^ d)
            idx = jnp.where(lane0, c, NUM_SEG + c)
            plsc.store_scatter(loc_s, [idx], jnp.where(lane0, bx, by),
                               mask=lane01)
            cntv = jnp.full((LANES,), cnt[0].astype(jnp.float32), jnp.float32)
            plsc.store_scatter(loc_c, [jnp.full((LANES,), c, jnp.int32)],
                               cntv, mask=lane0)
            accx[...] = zero16
            accy[...] = zero16
            cnt[0] = 0

        def advance_to(first):
            def tb(c, carry):
                flush(c)
                return carry

            lax.fori_loop(cur[0], first, tb, 0)
            cur[0] = first

        def do_vreg(bx, by, bi, roff):
            s = bi[pl.ds(roff, LANES)]
            s_first = s[0]
            s_last = s[LANES - 1]
            vx = bx[pl.ds(roff, LANES)]
            vy = by[pl.ds(roff, LANES)]
            c0 = cur[0]
            uni = (s_first == c0) & (s_last == c0)

            @pl.when(uni)
            def _fast():
                accx[...] = accx[...] + vx
                accy[...] = accy[...] + vy
                cnt[0] = cnt[0] + LANES

            @pl.when(jnp.logical_not(uni))
            def _slow():
                def cb(c, carry):
                    m = s == c
                    accx[...] = accx[...] + jnp.where(m, vx, 0.0)
                    accy[...] = accy[...] + jnp.where(m, vy, 0.0)
                    cnt[0] = cnt[0] + plsc.all_reduce_population_count(m)[0]

                    @pl.when(c < s_last)
                    def _():
                        flush(c)

                    return carry

                lax.fori_loop(c0, s_last + 1, cb, 0)
                cur[0] = s_last

        def tree_sum(vals):
            while len(vals) > 1:
                nxt = [vals[i] + vals[i + 1]
                       for i in range(0, len(vals) - 1, 2)]
                if len(vals) % 2:
                    nxt.append(vals[-1])
                vals = nxt
            return vals[0]

        def do_chunk(bx, by, bi, ch):
            roff = ch * cr
            first = bi[pl.ds(roff, LANES)][0]
            last = bi[pl.ds(roff + cr - LANES, LANES)][LANES - 1]
            uni = first == last

            @pl.when(uni)
            def _fastc():
                advance_to(first)
                accx[...] = accx[...] + tree_sum(
                    [bx[pl.ds(roff + LANES * k, LANES)] for k in range(kv)])
                accy[...] = accy[...] + tree_sum(
                    [by[pl.ds(roff + LANES * k, LANES)] for k in range(kv)])
                cnt[0] = cnt[0] + cr

            @pl.when(jnp.logical_not(uni))
            def _slowc():
                def vb(k, carry):
                    do_vreg(bx, by, bi, roff + k * LANES)
                    return carry

                lax.fori_loop(0, kv, vb, 0)

        def do_block(bx, by, bi, blk):
            @pl.when(blk == 0)
            def _():
                cur[0] = bi[pl.ds(0, LANES)][0]

            def chb(ch, carry):
                do_chunk(bx, by, bi, ch)
                return carry

            lax.fori_loop(0, nch, chb, 0)

        start(0, bufx0, bufy0, bufi0, sd0, si0)
        start(1, bufx1, bufy1, bufi1, sd1, si1)
        ring = ((bufx0, bufy0, bufi0, sd0, si0),
                (bufx1, bufy1, bufi1, sd1, si1))

        def bb(i, carry):
            for j in range(2):
                bx, by, bi, semd, semi = ring[j]
                blk = i * 2 + j
                wait(blk, bx, by, bi, semd, semi)
                do_block(bx, by, bi, blk)

                @pl.when(blk + 2 < nb)
                def _():
                    start(blk + 2, bx, by, bi, semd, semi)

            return carry

        lax.fori_loop(0, nb // 2, bb, 0)
        flush(cur[0])
        pltpu.sync_copy(loc_s, out_s.at[w])
        pltpu.sync_copy(loc_c, out_c.at[w])

    mesh = plsc.VectorSubcoreMesh(core_axis_name="c", subcore_axis_name="s")
    return pl.kernel(
        body,
        out_type=[
            jax.ShapeDtypeStruct((NW, 2 * NUM_SEG), jnp.float32),
            jax.ShapeDtypeStruct((NW, NUM_SEG), jnp.float32),
        ],
        mesh=mesh,
        compiler_params=pltpu.CompilerParams(needs_layout_passes=False),
        scratch_types=[
            pltpu.VMEM((br,), jnp.float32),
            pltpu.VMEM((br,), jnp.float32),
            pltpu.VMEM((br,), jnp.float32),
            pltpu.VMEM((br,), jnp.float32),
            pltpu.VMEM((br,), jnp.int32),
            pltpu.VMEM((br,), jnp.int32),
            pltpu.VMEM((2 * NUM_SEG,), jnp.float32),
            pltpu.VMEM((NUM_SEG,), jnp.float32),
            pltpu.VMEM((LANES,), jnp.float32),
            pltpu.VMEM((LANES,), jnp.float32),
            pltpu.SMEM((1,), jnp.int32),
            pltpu.SMEM((1,), jnp.int32),
            pltpu.SemaphoreType.DMA,
            pltpu.SemaphoreType.DMA,
            pltpu.SemaphoreType.DMA,
            pltpu.SemaphoreType.DMA,
        ],
    )


def _combine_body(s_ref, c_ref, o_ref):
    s = s_ref[...]
    tx = jnp.sum(s[:, :NUM_SEG], axis=0)
    ty = jnp.sum(s[:, NUM_SEG:], axis=0)
    cnt = jnp.sum(c_ref[...], axis=0)
    denom = jnp.maximum(cnt, 1.0)
    good = cnt > 0.0
    mx = jnp.where(good, tx / denom, 0.0)
    my = jnp.where(good, ty / denom, 0.0)
    o_ref[...] = jnp.stack([mx, my], axis=0)


def kernel(data, segment_ids):
    n = data.shape[0]
    sums, cnts = _sc_segment_partials(n)(data[:, 0], data[:, 1], segment_ids)
    out2 = pl.pallas_call(
        _combine_body,
        out_shape=jax.ShapeDtypeStruct((2, NUM_SEG), jnp.float32),
    )(sums, cnts)
    return out2.T


# speedup vs baseline: 1.1324x; 1.1324x over previous
import functools

import jax
import jax.numpy as jnp
from jax import lax
from jax.experimental import pallas as pl
from jax.experimental.pallas import tpu as pltpu
from jax.experimental.pallas import tpu_sc as plsc

NUM_SEG = 1024
NC = 2
NS = 16
NW = NC * NS
LANES = 16


def _lane_gather(x, idx):
    dnums = lax.GatherDimensionNumbers(
        offset_dims=(), collapsed_slice_dims=(0,), start_index_map=(0,))
    return lax.gather(x, idx[:, None], dnums, (1,),
                      mode=lax.GatherScatterMode.PROMISE_IN_BOUNDS)


def _sc_segment_partials(n_rows: int):
    rows_w = n_rows // NW
    assert rows_w * NW == n_rows
    nb = 20
    br = rows_w // nb
    assert br * nb == rows_w and br % 8 == 0
    cr = 400
    nch = br // cr
    assert nch * cr == br and cr % LANES == 0
    kv = cr // LANES

    def body(dxy, ids, out_s, out_c,
             bufx0, bufy0, bufx1, bufy1, bufi0, bufi1,
             loc_s, loc_c, accx, accy, cur, cnt,
             sd0, si0, sd1, si1):
        w = lax.axis_index("s") * NC + lax.axis_index("c")
        row0 = w * rows_w
        zero16 = jnp.zeros((LANES,), jnp.float32)
        iota = lax.iota(jnp.int32, LANES)
        lane0 = iota == 0
        lane01 = iota < 2

        def z_s(i, c):
            loc_s[pl.ds(i * LANES, LANES)] = zero16
            return c

        lax.fori_loop(0, (2 * NUM_SEG) // LANES, z_s, 0)

        def z_c(i, c):
            loc_c[pl.ds(i * LANES, LANES)] = zero16
            return c

        lax.fori_loop(0, NUM_SEG // LANES, z_c, 0)
        accx[...] = zero16
        accy[...] = zero16
        cnt[0] = 0
        cur[0] = 0

        def start(blk, bx, by, bi, semd, semi):
            rx = pl.ds(row0 + blk * br, br)
            ry = pl.ds(n_rows + row0 + blk * br, br)
            pltpu.async_copy(dxy.at[rx], bx, semd)
            pltpu.async_copy(dxy.at[ry], by, semd)
            pltpu.async_copy(ids.at[rx], bi, semi)

        def wait(blk, bx, by, bi, semd, semi):
            rx = pl.ds(row0 + blk * br, br)
            ry = pl.ds(n_rows + row0 + blk * br, br)
            pltpu.make_async_copy(dxy.at[rx], bx, semd).wait()
            pltpu.make_async_copy(dxy.at[ry], by, semd).wait()
            pltpu.make_async_copy(ids.at[rx], bi, semi).wait()

        def flush(c):
            bx = accx[...]
            by = accy[...]
            for d in (1, 2, 4, 8):
                bx = bx + _lane_gather(bx, iota ^ d)
                by = by + _lane_gather(by, iota ^ d)
            idx = jnp.where(lane0, c, NUM_SEG + c)
            plsc.store_scatter(loc_s, [idx], jnp.where(lane0, bx, by),
                               mask=lane01)
            cntv = jnp.full((LANES,), cnt[0].astype(jnp.float32), jnp.float32)
            plsc.store_scatter(loc_c, [jnp.full((LANES,), c, jnp.int32)],
                               cntv, mask=lane0)
            accx[...] = zero16
            accy[...] = zero16
            cnt[0] = 0

        def advance_to(first):
            def tb(c, carry):
                flush(c)
                return carry

            lax.fori_loop(cur[0], first, tb, 0)
            cur[0] = first

        def do_vreg(bx, by, bi, roff):
            s = bi[pl.ds(roff, LANES)]
            s_first = s[0]
            s_last = s[LANES - 1]
            vx = bx[pl.ds(roff, LANES)]
            vy = by[pl.ds(roff, LANES)]
            c0 = cur[0]
            uni = (s_first == c0) & (s_last == c0)

            @pl.when(uni)
            def _fast():
                accx[...] = accx[...] + vx
                accy[...] = accy[...] + vy
                cnt[0] = cnt[0] + LANES

            @pl.when(jnp.logical_not(uni))
            def _slow():
                def cb(c, carry):
                    m = s == c
                    accx[...] = accx[...] + jnp.where(m, vx, 0.0)
                    accy[...] = accy[...] + jnp.where(m, vy, 0.0)
                    cnt[0] = cnt[0] + plsc.all_reduce_population_count(m)[0]

                    @pl.when(c < s_last)
                    def _():
                        flush(c)

                    return carry

                lax.fori_loop(c0, s_last + 1, cb, 0)
                cur[0] = s_last

        def tree_sum(vals):
            while len(vals) > 1:
                nxt = [vals[i] + vals[i + 1]
                       for i in range(0, len(vals) - 1, 2)]
                if len(vals) % 2:
                    nxt.append(vals[-1])
                vals = nxt
            return vals[0]

        def do_chunk(bx, by, bi, ch):
            roff = ch * cr
            first = bi[pl.ds(roff, LANES)][0]
            last = bi[pl.ds(roff + cr - LANES, LANES)][LANES - 1]
            uni = first == last

            @pl.when(uni)
            def _fastc():
                advance_to(first)
                accx[...] = accx[...] + tree_sum(
                    [bx[pl.ds(roff + LANES * k, LANES)] for k in range(kv)])
                accy[...] = accy[...] + tree_sum(
                    [by[pl.ds(roff + LANES * k, LANES)] for k in range(kv)])
                cnt[0] = cnt[0] + cr

            @pl.when(jnp.logical_not(uni))
            def _slowc():
                def vb(k, carry):
                    do_vreg(bx, by, bi, roff + k * LANES)
                    return carry

                lax.fori_loop(0, kv, vb, 0)

        def do_block(bx, by, bi, blk):
            @pl.when(blk == 0)
            def _():
                cur[0] = bi[pl.ds(0, LANES)][0]

            def chb(ch, carry):
                do_chunk(bx, by, bi, ch)
                return carry

            lax.fori_loop(0, nch, chb, 0)

        start(0, bufx0, bufy0, bufi0, sd0, si0)
        start(1, bufx1, bufy1, bufi1, sd1, si1)
        ring = ((bufx0, bufy0, bufi0, sd0, si0),
                (bufx1, bufy1, bufi1, sd1, si1))

        def bb(i, carry):
            for j in range(2):
                bx, by, bi, semd, semi = ring[j]
                blk = i * 2 + j
                wait(blk, bx, by, bi, semd, semi)
                do_block(bx, by, bi, blk)

                @pl.when(blk + 2 < nb)
                def _():
                    start(blk + 2, bx, by, bi, semd, semi)

            return carry

        lax.fori_loop(0, nb // 2, bb, 0)
        flush(cur[0])
        pltpu.sync_copy(loc_s, out_s.at[w])
        pltpu.sync_copy(loc_c, out_c.at[w])

    mesh = plsc.VectorSubcoreMesh(core_axis_name="c", subcore_axis_name="s")
    return pl.kernel(
        body,
        out_type=[
            jax.ShapeDtypeStruct((NW, 2 * NUM_SEG), jnp.float32),
            jax.ShapeDtypeStruct((NW, NUM_SEG), jnp.float32),
        ],
        mesh=mesh,
        compiler_params=pltpu.CompilerParams(needs_layout_passes=False),
        scratch_types=[
            pltpu.VMEM((br,), jnp.float32),
            pltpu.VMEM((br,), jnp.float32),
            pltpu.VMEM((br,), jnp.float32),
            pltpu.VMEM((br,), jnp.float32),
            pltpu.VMEM((br,), jnp.int32),
            pltpu.VMEM((br,), jnp.int32),
            pltpu.VMEM((2 * NUM_SEG,), jnp.float32),
            pltpu.VMEM((NUM_SEG,), jnp.float32),
            pltpu.VMEM((LANES,), jnp.float32),
            pltpu.VMEM((LANES,), jnp.float32),
            pltpu.SMEM((1,), jnp.int32),
            pltpu.SMEM((1,), jnp.int32),
            pltpu.SemaphoreType.DMA,
            pltpu.SemaphoreType.DMA,
            pltpu.SemaphoreType.DMA,
            pltpu.SemaphoreType.DMA,
        ],
    )


def _combine_body(s_ref, c_ref, o_ref):
    s = s_ref[...]
    tx = jnp.sum(s[:, :NUM_SEG], axis=0)
    ty = jnp.sum(s[:, NUM_SEG:], axis=0)
    cnt = jnp.sum(c_ref[...], axis=0)
    denom = jnp.maximum(cnt, 1.0)
    good = cnt > 0.0
    mx = jnp.where(good, tx / denom, 0.0)
    my = jnp.where(good, ty / denom, 0.0)
    o_ref[...] = jnp.stack([mx, my], axis=0)


def kernel(data, segment_ids):
    n = data.shape[0]
    sums, cnts = _sc_segment_partials(n)(data.T.reshape(-1), segment_ids)
    out2 = pl.pallas_call(
        _combine_body,
        out_shape=jax.ShapeDtypeStruct((2, NUM_SEG), jnp.float32),
    )(sums, cnts)
    return out2.T


# speedup vs baseline: 1.6358x; 1.4445x over previous
import functools

import jax
import jax.numpy as jnp
from jax import lax
from jax.experimental import pallas as pl
from jax.experimental.pallas import tpu as pltpu
from jax.experimental.pallas import tpu_sc as plsc

NUM_SEG = 1024
NC = 2
NS = 16
NW = NC * NS
LANES = 16


def _lane_gather(x, idx):
    dnums = lax.GatherDimensionNumbers(
        offset_dims=(), collapsed_slice_dims=(0,), start_index_map=(0,))
    return lax.gather(x, idx[:, None], dnums, (1,),
                      mode=lax.GatherScatterMode.PROMISE_IN_BOUNDS)


def _sc_segment_partials(n_rows: int):
    br = 5120
    nblk = n_rows // br
    assert nblk * br == n_rows
    base = nblk // NW
    rem = nblk % NW
    n_iter = (base + 2) // 2
    cr = 320
    nch = br // cr
    assert nch * cr == br and cr % LANES == 0
    kv = cr // LANES

    def body(dxy, ids, out_s, out_c,
             bufd0, bufd1, bufi0, bufi1,
             loc_s, loc_c, accx, accy, cur, cnt,
             sd0, si0, sd1, si1):
        w = lax.axis_index("s") * NC + lax.axis_index("c")
        nb_w = jnp.where(w < rem, base + 1, base)
        b0 = jnp.where(w < rem, w * (base + 1),
                       rem * (base + 1) + (w - rem) * base)
        zero16 = jnp.zeros((LANES,), jnp.float32)
        iota = lax.iota(jnp.int32, LANES)
        lane0 = iota == 0
        lane01 = iota < 2

        def z_s(i, c):
            loc_s[pl.ds(i * LANES, LANES)] = zero16
            return c

        lax.fori_loop(0, (2 * NUM_SEG) // LANES, z_s, 0)

        def z_c(i, c):
            loc_c[pl.ds(i * LANES, LANES)] = zero16
            return c

        lax.fori_loop(0, NUM_SEG // LANES, z_c, 0)
        accx[...] = zero16
        accy[...] = zero16
        cnt[0] = 0
        cur[0] = 0

        def start(gblk, bd, bi, semd, semi):
            r = pl.ds(gblk * br, br)
            pltpu.async_copy(dxy.at[:, r], bd, semd)
            pltpu.async_copy(ids.at[r], bi, semi)

        def wait(gblk, bd, bi, semd, semi):
            r = pl.ds(gblk * br, br)
            pltpu.make_async_copy(dxy.at[:, r], bd, semd).wait()
            pltpu.make_async_copy(ids.at[r], bi, semi).wait()

        def flush(c):
            bx = accx[...]
            by = accy[...]
            for d in (1, 2, 4, 8):
                bx = bx + _lane_gather(bx, iota ^ d)
                by = by + _lane_gather(by, iota ^ d)
            idx = jnp.where(lane0, c, NUM_SEG + c)
            plsc.store_scatter(loc_s, [idx], jnp.where(lane0, bx, by),
                               mask=lane01)
            cntv = jnp.full((LANES,), cnt[0].astype(jnp.float32), jnp.float32)
            plsc.store_scatter(loc_c, [jnp.full((LANES,), c, jnp.int32)],
                               cntv, mask=lane0)
            accx[...] = zero16
            accy[...] = zero16
            cnt[0] = 0

        def advance_to(first):
            def tb(c, carry):
                flush(c)
                return carry

            lax.fori_loop(cur[0], first, tb, 0)
            cur[0] = first

        def do_vreg(bd, bi, roff):
            s = bi[pl.ds(roff, LANES)]
            s_first = s[0]
            s_last = s[LANES - 1]
            vx = bd[0, pl.ds(roff, LANES)]
            vy = bd[1, pl.ds(roff, LANES)]
            c0 = cur[0]
            uni = (s_first == c0) & (s_last == c0)

            @pl.when(uni)
            def _fast():
                accx[...] = accx[...] + vx
                accy[...] = accy[...] + vy
                cnt[0] = cnt[0] + LANES

            @pl.when(jnp.logical_not(uni))
            def _slow():
                def cb(c, carry):
                    m = s == c
                    accx[...] = accx[...] + jnp.where(m, vx, 0.0)
                    accy[...] = accy[...] + jnp.where(m, vy, 0.0)
                    cnt[0] = cnt[0] + plsc.all_reduce_population_count(m)[0]

                    @pl.when(c < s_last)
                    def _():
                        flush(c)

                    return carry

                lax.fori_loop(c0, s_last + 1, cb, 0)
                cur[0] = s_last

        def tree_sum(vals):
            while len(vals) > 1:
                nxt = [vals[i] + vals[i + 1]
                       for i in range(0, len(vals) - 1, 2)]
                if len(vals) % 2:
                    nxt.append(vals[-1])
                vals = nxt
            return vals[0]

        def do_chunk(bd, bi, ch):
            roff = ch * cr
            first = bi[pl.ds(roff, LANES)][0]
            last = bi[pl.ds(roff + cr - LANES, LANES)][LANES - 1]
            uni = first == last

            @pl.when(uni)
            def _fastc():
                advance_to(first)
                accx[...] = accx[...] + tree_sum(
                    [bd[0, pl.ds(roff + LANES * k, LANES)] for k in range(kv)])
                accy[...] = accy[...] + tree_sum(
                    [bd[1, pl.ds(roff + LANES * k, LANES)] for k in range(kv)])
                cnt[0] = cnt[0] + cr

            @pl.when(jnp.logical_not(uni))
            def _slowc():
                def vb(k, carry):
                    do_vreg(bd, bi, roff + k * LANES)
                    return carry

                lax.fori_loop(0, kv, vb, 0)

        def do_block(bd, bi, blk):
            @pl.when(blk == 0)
            def _():
                cur[0] = bi[pl.ds(0, LANES)][0]

            def chb(ch, carry):
                do_chunk(bd, bi, ch)
                return carry

            lax.fori_loop(0, nch, chb, 0)

        start(b0, bufd0, bufi0, sd0, si0)
        start(b0 + 1, bufd1, bufi1, sd1, si1)
        ring = ((bufd0, bufi0, sd0, si0),
                (bufd1, bufi1, sd1, si1))

        def bb(i, carry):
            for j in range(2):
                bd, bi, semd, semi = ring[j]
                blk = i * 2 + j

                @pl.when(blk < nb_w)
                def _():
                    wait(b0 + blk, bd, bi, semd, semi)
                    do_block(bd, bi, blk)

                    @pl.when(blk + 2 < nb_w)
                    def _():
                        start(b0 + blk + 2, bd, bi, semd, semi)

            return carry

        lax.fori_loop(0, n_iter, bb, 0)
        flush(cur[0])
        pltpu.sync_copy(loc_s, out_s.at[w])
        pltpu.sync_copy(loc_c, out_c.at[w])

    mesh = plsc.VectorSubcoreMesh(core_axis_name="c", subcore_axis_name="s")
    return pl.kernel(
        body,
        out_type=[
            jax.ShapeDtypeStruct((NW, 2 * NUM_SEG), jnp.float32),
            jax.ShapeDtypeStruct((NW, NUM_SEG), jnp.float32),
        ],
        mesh=mesh,
        compiler_params=pltpu.CompilerParams(needs_layout_passes=False),
        scratch_types=[
            pltpu.VMEM((2, br), jnp.float32),
            pltpu.VMEM((2, br), jnp.float32),
            pltpu.VMEM((br,), jnp.int32),
            pltpu.VMEM((br,), jnp.int32),
            pltpu.VMEM((2 * NUM_SEG,), jnp.float32),
            pltpu.VMEM((NUM_SEG,), jnp.float32),
            pltpu.VMEM((LANES,), jnp.float32),
            pltpu.VMEM((LANES,), jnp.float32),
            pltpu.SMEM((1,), jnp.int32),
            pltpu.SMEM((1,), jnp.int32),
            pltpu.SemaphoreType.DMA,
            pltpu.SemaphoreType.DMA,
            pltpu.SemaphoreType.DMA,
            pltpu.SemaphoreType.DMA,
        ],
    )


def _combine_body(s_ref, c_ref, o_ref):
    s = s_ref[...]
    tx = jnp.sum(s[:, :NUM_SEG], axis=0)
    ty = jnp.sum(s[:, NUM_SEG:], axis=0)
    cnt = jnp.sum(c_ref[...], axis=0)
    denom = jnp.maximum(cnt, 1.0)
    good = cnt > 0.0
    mx = jnp.where(good, tx / denom, 0.0)
    my = jnp.where(good, ty / denom, 0.0)
    o_ref[...] = jnp.stack([mx, my], axis=0)


def kernel(data, segment_ids):
    n = data.shape[0]
    sums, cnts = _sc_segment_partials(n)(data.T, segment_ids)
    out2 = pl.pallas_call(
        _combine_body,
        out_shape=jax.ShapeDtypeStruct((2, NUM_SEG), jnp.float32),
    )(sums, cnts)
    return out2.T
